# baseline (speedup 1.0000x reference)
import jax
import jax.numpy as jnp
from jax.experimental import pallas as pl
from jax.experimental.pallas import tpu as pltpu

_M_BLK = 512
_F_BLK = 2048


def _ffn_block_kernel(x_ref, w1_ref, w2_ref, o_ref):
    f = pl.program_id(1)

    @pl.when(f == 0)
    def _init():
        o_ref[...] = jnp.zeros_like(o_ref)

    zf = jax.lax.dot_general(
        x_ref[...], w1_ref[...],
        (((1,), (1,)), ((), ())),
        preferred_element_type=jnp.float32,
    )
    h = (0.5 * zf * (1.0 + jax.lax.erf(zf * 0.7071067811865476))).astype(
        jnp.bfloat16)
    o_ref[...] += jax.lax.dot_general(
        h, w2_ref[...],
        (((1,), (1,)), ((), ())),
        preferred_element_type=jnp.float32,
    )


def kernel(x, Wr, W1, W2):
    del Wr
    B, T, D = x.shape
    N = B * T
    x_bf = x.reshape(N, D).astype(jnp.bfloat16)
    w1_bf = W1.astype(jnp.bfloat16)
    w2_bf = W2.astype(jnp.bfloat16)
    F = W1.shape[0]

    out = pl.pallas_call(
        _ffn_block_kernel,
        grid=(N // _M_BLK, F // _F_BLK),
        in_specs=[
            pl.BlockSpec((_M_BLK, D), lambda m, f: (m, 0)),
            pl.BlockSpec((_F_BLK, D), lambda m, f: (f, 0)),
            pl.BlockSpec((D, _F_BLK), lambda m, f: (0, f)),
        ],
        out_specs=pl.BlockSpec((_M_BLK, D), lambda m, f: (m, 0)),
        out_shape=jax.ShapeDtypeStruct((N, D), jnp.float32),
        compiler_params=pltpu.CompilerParams(
            dimension_semantics=("parallel", "arbitrary"),
        ),
    )(x_bf, w1_bf, w2_bf)
    return out.reshape(B, T, D)


# speedup vs baseline: 1.0488x; 1.0488x over previous
import jax
import jax.numpy as jnp
from jax.experimental import pallas as pl
from jax.experimental.pallas import tpu as pltpu

_M_BLK = 512
_F_BLK = 2048


def _ffn_block_kernel(x_ref, w1_ref, w2_ref, o_ref):
    f = pl.program_id(1)

    @pl.when(f == 0)
    def _init():
        o_ref[...] = jnp.zeros_like(o_ref)

    zf = jax.lax.dot_general(
        x_ref[...].astype(jnp.bfloat16), w1_ref[...],
        (((1,), (1,)), ((), ())),
        preferred_element_type=jnp.float32,
    )
    h = (0.5 * zf * (1.0 + jax.lax.erf(zf * 0.7071067811865476))).astype(
        jnp.bfloat16)
    o_ref[...] += jax.lax.dot_general(
        h, w2_ref[...],
        (((1,), (1,)), ((), ())),
        preferred_element_type=jnp.float32,
    )


def kernel(x, Wr, W1, W2):
    del Wr
    B, T, D = x.shape
    N = B * T
    x_flat = x.reshape(N, D)
    w1_bf = W1.astype(jnp.bfloat16)
    w2_bf = W2.astype(jnp.bfloat16)
    F = W1.shape[0]

    out = pl.pallas_call(
        _ffn_block_kernel,
        grid=(N // _M_BLK, F // _F_BLK),
        in_specs=[
            pl.BlockSpec((_M_BLK, D), lambda m, f: (m, 0)),
            pl.BlockSpec((_F_BLK, D), lambda m, f: (f, 0)),
            pl.BlockSpec((D, _F_BLK), lambda m, f: (0, f)),
        ],
        out_specs=pl.BlockSpec((_M_BLK, D), lambda m, f: (m, 0)),
        out_shape=jax.ShapeDtypeStruct((N, D), jnp.float32),
        compiler_params=pltpu.CompilerParams(
            dimension_semantics=("parallel", "arbitrary"),
        ),
    )(x_flat, w1_bf, w2_bf)
    return out.reshape(B, T, D)
